# Initial kernel scaffold; baseline (speedup 1.0000x reference)
import jax
import jax.numpy as jnp
from jax.experimental import pallas as pl


def kernel(x, W0, b0, W1, b1, W2, b2, Wd0, bd0, Wd1, bd1, Wd2, bd2, k):
    raise NotImplementedError("write your pallas kernel here")


# speedup vs baseline: 7.5460x; 7.5460x over previous
import functools

import jax
import jax.numpy as jnp
from jax import lax
from jax.experimental import pallas as pl
from jax.experimental.pallas import tpu as pltpu
from jax.experimental.pallas import tpu_sc as plsc

N = 8192
KNN = 20
BQ = 256
NW = 32
QW = N // NW
QCH = 4
CH = QW // QCH
ROWS = QCH * KNN
BIGF = 3.0e38
BIGI = 2 ** 30


def _knn_body(x_ref, xt_ref, idx_ref, d_scr):
    i = pl.program_id(0)
    xb = x_ref[...]
    xt = xt_ref[...]
    dot = jnp.dot(xb, xt, preferred_element_type=jnp.float32)
    sqb = jnp.sum(xb * xb, axis=1, keepdims=True)
    sqc = jnp.sum(xt * xt, axis=0, keepdims=True)
    d_scr[...] = (sqb + sqc) - 2.0 * dot

    rows = lax.broadcasted_iota(jnp.int32, (BQ, BQ), 0)
    cols = lax.broadcasted_iota(jnp.int32, (BQ, BQ), 1)
    eyeb = jnp.where(rows == cols, jnp.float32(1e10), jnp.float32(0.0))
    blk = d_scr[:, pl.ds(i * BQ, BQ)]
    d_scr[:, pl.ds(i * BQ, BQ)] = blk + eyeb

    lane = lax.broadcasted_iota(jnp.int32, (BQ, 128), 1)

    def body(t, acc):
        d = d_scr[...]
        io = lax.broadcasted_iota(jnp.int32, (BQ, N), 1)
        m = jnp.min(d, axis=1, keepdims=True)
        cand = jnp.where(d == m, io, BIGI)
        pos = jnp.min(cand, axis=1, keepdims=True)
        d_scr[...] = jnp.where(cand == pos, BIGF, d)
        return jnp.where(lane == t, pos, acc)

    acc = lax.fori_loop(0, KNN, body, jnp.zeros((BQ, 128), jnp.int32))
    idx_ref[...] = acc[:, :KNN]


def _knn(x):
    xt = x.T
    return pl.pallas_call(
        _knn_body,
        grid=(N // BQ,),
        in_specs=[
            pl.BlockSpec((BQ, 3), lambda i: (i, 0)),
            pl.BlockSpec((3, N), lambda i: (0, 0)),
        ],
        out_specs=pl.BlockSpec((BQ, KNN), lambda i: (i, 0)),
        out_shape=jax.ShapeDtypeStruct((N, KNN), jnp.int32),
        scratch_shapes=[
            pltpu.VMEM((BQ, N), jnp.float32),
        ],
    )(x, xt)


def _gmax_body(F, b_hbm, idx_hbm, out_hbm, idx_v, rows_v, out_v, sem0, sem1):
    sems = (sem0, sem1)
    wid = lax.axis_index("s") * 2 + lax.axis_index("c")
    pltpu.sync_copy(idx_hbm.at[wid], idx_v)

    def gstart(c, b):
        pltpu.async_copy(b_hbm.at[idx_v.at[c]], rows_v.at[b], sems[b])

    def gwait(c, b):
        pltpu.make_async_copy(b_hbm.at[idx_v.at[c]], rows_v.at[b],
                              sems[b]).wait()

    def reduce_chunk(c, b):
        for q in range(QCH):
            accs = [rows_v[b, q * KNN, pl.ds(fc * 16, 16)]
                    for fc in range(F // 16)]

            def jbody(j, accs):
                return tuple(
                    jnp.maximum(a, rows_v[b, q * KNN + j, pl.ds(fc * 16, 16)])
                    for fc, a in enumerate(accs))

            accs = lax.fori_loop(1, KNN, jbody, tuple(accs))
            for fc in range(F // 16):
                out_v[c * QCH + q, pl.ds(fc * 16, 16)] = accs[fc]

    gstart(0, 0)
    gstart(1, 1)

    def body(g, _):
        c0 = 2 * g
        gwait(c0, 0)
        reduce_chunk(c0, 0)

        @pl.when(c0 + 2 < CH)
        def _():
            gstart(c0 + 2, 0)

        gwait(c0 + 1, 1)
        reduce_chunk(c0 + 1, 1)

        @pl.when(c0 + 3 < CH)
        def _():
            gstart(c0 + 3, 1)

        return 0

    lax.fori_loop(0, CH // 2, body, 0)
    pltpu.sync_copy(out_v, out_hbm.at[pl.ds(wid * QW, QW)])


def _gmax(bmat, idx3, F):
    mesh = plsc.VectorSubcoreMesh(core_axis_name="c", subcore_axis_name="s")
    fn = pl.kernel(
        functools.partial(_gmax_body, F),
        mesh=mesh,
        out_type=jax.ShapeDtypeStruct((N, F), jnp.float32),
        scratch_types=[
            pltpu.VMEM((CH, ROWS), jnp.int32),
            pltpu.VMEM((2, ROWS, F), jnp.float32),
            pltpu.VMEM((QW, F), jnp.float32),
            pltpu.SemaphoreType.DMA,
            pltpu.SemaphoreType.DMA,
        ],
    )
    return fn(bmat, idx3)


def _cb0_body(x_ref, w_ref, b_ref, c_ref, bm_ref):
    h = x_ref[...]
    w = w_ref[...]
    fin = h.shape[1]
    wa = w[:fin] - w[fin:]
    wb = w[fin:]
    c_ref[...] = jnp.dot(h, wa, preferred_element_type=jnp.float32) + b_ref[...]
    bm_ref[...] = jnp.dot(h, wb, preferred_element_type=jnp.float32)


def _cb_body(cp_ref, mp_ref, w_ref, b_ref, c_ref, bm_ref):
    h = jnp.maximum(cp_ref[...] + mp_ref[...], 0.0)
    w = w_ref[...]
    fin = h.shape[1]
    wa = w[:fin] - w[fin:]
    wb = w[fin:]
    c_ref[...] = jnp.dot(h, wa, preferred_element_type=jnp.float32) + b_ref[...]
    bm_ref[...] = jnp.dot(h, wb, preferred_element_type=jnp.float32)


def _cb0(x, w, b):
    fout = w.shape[1]
    return pl.pallas_call(
        _cb0_body,
        out_shape=(jax.ShapeDtypeStruct((N, fout), jnp.float32),
                   jax.ShapeDtypeStruct((N, fout), jnp.float32)),
    )(x, w, b.reshape(1, -1))


def _cb(cprev, mprev, w, b):
    fout = w.shape[1]
    return pl.pallas_call(
        _cb_body,
        out_shape=(jax.ShapeDtypeStruct((N, fout), jnp.float32),
                   jax.ShapeDtypeStruct((N, fout), jnp.float32)),
    )(cprev, mprev, w, b.reshape(1, -1))


def _dec_body(c_ref, m_ref, w0_ref, b0_ref, w1_ref, b1_ref, w2_ref, b2_ref,
              o_ref):
    h = jnp.maximum(c_ref[...] + m_ref[...], 0.0)
    h = jnp.maximum(
        jnp.dot(h, w0_ref[...], preferred_element_type=jnp.float32)
        + b0_ref[...], 0.0)
    h = jnp.maximum(
        jnp.dot(h, w1_ref[...], preferred_element_type=jnp.float32)
        + b1_ref[...], 0.0)
    o_ref[...] = (jnp.dot(h, w2_ref[...], preferred_element_type=jnp.float32)
                  + b2_ref[...])


def _dec(c2, m2, wd0, bd0, wd1, bd1, wd2, bd2):
    return pl.pallas_call(
        _dec_body,
        out_shape=jax.ShapeDtypeStruct((N, wd2.shape[1]), jnp.float32),
    )(c2, m2, wd0, bd0.reshape(1, -1), wd1, bd1.reshape(1, -1),
      wd2, bd2.reshape(1, -1))


def kernel(x, W0, b0, W1, b1, W2, b2, Wd0, bd0, Wd1, bd1, Wd2, bd2, k):
    idx = _knn(x)
    idx = idx + (jnp.asarray(k) - KNN).astype(idx.dtype)
    idx3 = idx.reshape(NW, CH, ROWS)

    W0p = jnp.concatenate([W0, jnp.zeros_like(W0)], axis=1)
    c0, B0 = _cb0(x, W0p, jnp.concatenate([b0, jnp.zeros_like(b0)]))
    c0 = c0[:, :64]
    M0 = _gmax(B0, idx3, 128)[:, :64]
    c1, B1 = _cb(c0, M0, W1, b1)
    M1 = _gmax(B1, idx3, 128)
    c2, B2 = _cb(c1, M1, W2, b2)
    M2 = _gmax(B2, idx3, 256)
    return _dec(c2, M2, Wd0, bd0, Wd1, bd1, Wd2, bd2)
